# baseline (speedup 1.0000x reference)
import functools

import jax
import jax.numpy as jnp
from jax.experimental import pallas as pl
from jax.experimental.pallas import tpu as pltpu

TOP_K = 8
NUM_EXPERTS = 64
EMBED_DIM = 4096
BLOCK_T = 1024

_KS0 = 0
_KS1 = 42
_KS2 = 0x1BD11BDA ^ _KS0 ^ _KS1
_ROTS = ((13, 15, 26, 6), (17, 29, 16, 24))
_INJECT = ((_KS1, _KS2, 1), (_KS2, _KS0, 2), (_KS0, _KS1, 3),
           (_KS1, _KS2, 4), (_KS2, _KS0, 5))


def _uniform_block(base, shape):
    row = jax.lax.broadcasted_iota(jnp.int32, shape, 0)
    col = jax.lax.broadcasted_iota(jnp.int32, shape, 1)
    flat = (base + row * shape[1] + col).astype(jnp.uint32)
    x0 = jnp.full(shape, jnp.uint32(_KS0), jnp.uint32)
    x1 = flat + jnp.uint32(_KS1)
    for blk in range(5):
        for r in _ROTS[blk % 2]:
            x0 = x0 + x1
            x1 = (x1 << r) | (x1 >> (32 - r))
            x1 = x1 ^ x0
        ka, kb, c = _INJECT[blk]
        x0 = x0 + jnp.uint32(ka)
        x1 = x1 + jnp.uint32(kb + c)
    bits = x0 ^ x1
    return jax.lax.bitcast_convert_type(
        (bits >> 9) | jnp.uint32(0x3F800000), jnp.float32
    ) - 1.0


def _router_block(x_ref, w_ref, b_ref, out_ref, idx_ref):
    x = x_ref[...]
    w = w_ref[...]
    acc = jnp.dot(x, w, preferred_element_type=jnp.float32) + b_ref[...]
    logits = acc[:, :NUM_EXPERTS]
    noise_logits = acc[:, NUM_EXPERTS:]
    sp = jnp.maximum(noise_logits, 0.0) + jnp.log1p(jnp.exp(-jnp.abs(noise_logits)))
    t = logits.shape[0]
    u = _uniform_block(pl.program_id(0) * (BLOCK_T * NUM_EXPERTS), (t, NUM_EXPERTS))
    noisy = logits + u * sp

    iota_f = jax.lax.broadcasted_iota(jnp.int32, (t, NUM_EXPERTS), 1).astype(
        jnp.float32
    )
    work = noisy
    selected = jnp.zeros((t, NUM_EXPERTS), dtype=jnp.bool_)
    idx_cols = []
    top1 = None
    for j in range(TOP_K):
        m = jnp.max(work, axis=1, keepdims=True)
        if j == 0:
            top1 = m
        idx = jnp.min(
            jnp.where(work == m, iota_f, float(NUM_EXPERTS)), axis=1, keepdims=True
        )
        idx_cols.append(idx)
        hit = iota_f == idx
        selected = jnp.logical_or(selected, hit)
        work = jnp.where(hit, -jnp.inf, work)

    e = jnp.where(selected, jnp.exp(noisy - top1), 0.0)
    out_ref[...] = e * (1.0 / jnp.sum(e, axis=1, keepdims=True))
    idx_ref[...] = jnp.concatenate(idx_cols, axis=1).astype(jnp.int32)


@functools.partial(jax.jit, static_argnames=())
def _run(x, w_cat, b_cat):
    n_tok = x.shape[0]
    grid = (n_tok // BLOCK_T,)
    out, idx = pl.pallas_call(
        _router_block,
        grid=grid,
        in_specs=[
            pl.BlockSpec((BLOCK_T, EMBED_DIM), lambda i: (i, 0)),
            pl.BlockSpec((EMBED_DIM, 2 * NUM_EXPERTS), lambda i: (0, 0)),
            pl.BlockSpec((1, 2 * NUM_EXPERTS), lambda i: (0, 0)),
        ],
        out_specs=[
            pl.BlockSpec((BLOCK_T, NUM_EXPERTS), lambda i: (i, 0)),
            pl.BlockSpec((BLOCK_T, TOP_K), lambda i: (i, 0)),
        ],
        out_shape=[
            jax.ShapeDtypeStruct((n_tok, NUM_EXPERTS), jnp.float32),
            jax.ShapeDtypeStruct((n_tok, TOP_K), jnp.int32),
        ],
    )(x, w_cat, b_cat)
    return out, idx


def kernel(mh_output, W_route, b_route, W_noise, b_noise):
    b, s, d = mh_output.shape
    x = mh_output.reshape(b * s, d)
    w_cat = jnp.concatenate([W_route, W_noise], axis=1)
    b_cat = jnp.concatenate([b_route, b_noise], axis=0).reshape(1, 2 * NUM_EXPERTS)
    out, idx = _run(x, w_cat, b_cat)
    return out.reshape(b, s, NUM_EXPERTS), idx.reshape(b, s, TOP_K)
